# baseline (speedup 1.0000x reference)
import jax
import jax.numpy as jnp
from jax.experimental import pallas as pl
from jax.experimental.pallas import tpu as pltpu

_K = 256.0
_B = 64
_T = 32768
_CH = 16
_N = _B // _CH
_IN_BUFS = 4
_OUT_BUFS = 3


def _compute(x, inv_temp):
    y = jax.nn.sigmoid(x * inv_temp)
    budget = jnp.clip(jnp.sum(y, axis=1, keepdims=True), 1e-6, None)
    y = y * jnp.minimum(_K / budget, 1.0)
    for d in (1, 2):
        shifted = pltpu.roll(y, shift=_T - d, axis=1)
        y = y * jnp.minimum(2.0 / (1.0 + y + shifted), 1.0)
    col = jax.lax.broadcasted_iota(jnp.int32, y.shape, 1)
    return jnp.where(col == 0, 0.0, y)


def _body(scale_ref, x_hbm, o_hbm, xb, ob, in_sems, out_sems):
    inv_temp = scale_ref[0]

    def in_copy(i, slot):
        return pltpu.make_async_copy(
            x_hbm.at[pl.ds(i * _CH, _CH)], xb.at[slot], in_sems.at[slot])

    def out_copy(i, slot):
        return pltpu.make_async_copy(
            ob.at[slot], o_hbm.at[pl.ds(i * _CH, _CH)], out_sems.at[slot])

    for i in range(min(_IN_BUFS, _N)):
        in_copy(i, i).start()

    for i in range(_N):
        islot = i % _IN_BUFS
        oslot = i % _OUT_BUFS
        if i >= _OUT_BUFS:
            out_copy(i - _OUT_BUFS, oslot).wait()
        in_copy(i, islot).wait()
        ob[oslot] = _compute(xb[islot], inv_temp)
        out_copy(i, oslot).start()
        nxt = i + _IN_BUFS
        if nxt < _N:
            in_copy(nxt, islot).start()

    for i in range(_N - min(_OUT_BUFS, _N), _N):
        out_copy(i, i % _OUT_BUFS).wait()


@jax.jit
def kernel(scores, log_temperature):
    temp = jnp.clip(jnp.exp(log_temperature), 0.1, 10.0)
    inv_temp = (1.0 / temp).reshape(1).astype(jnp.float32)
    return pl.pallas_call(
        _body,
        in_specs=[
            pl.BlockSpec(memory_space=pltpu.SMEM),
            pl.BlockSpec(memory_space=pltpu.HBM),
        ],
        out_specs=pl.BlockSpec(memory_space=pltpu.HBM),
        out_shape=jax.ShapeDtypeStruct((_B, _T), jnp.float32),
        scratch_shapes=[
            pltpu.VMEM((_IN_BUFS, _CH, _T), jnp.float32),
            pltpu.VMEM((_OUT_BUFS, _CH, _T), jnp.float32),
            pltpu.SemaphoreType.DMA((_IN_BUFS,)),
            pltpu.SemaphoreType.DMA((_OUT_BUFS,)),
        ],
    )(inv_temp, scores)


# speedup vs baseline: 1.0061x; 1.0061x over previous
import jax
import jax.numpy as jnp
from jax.experimental import pallas as pl
from jax.experimental.pallas import tpu as pltpu

_K = 256.0
_B = 64
_T = 32768
_CH = 16
_N = _B // _CH
_IN_BUFS = 3
_OUT_BUFS = 2


def _compute(x, inv_temp):
    y = jax.nn.sigmoid(x * inv_temp)
    budget = jnp.clip(jnp.sum(y, axis=1, keepdims=True), 1e-6, None)
    y = y * jnp.minimum(_K / budget, 1.0)
    for d in (1, 2):
        shifted = pltpu.roll(y, shift=_T - d, axis=1)
        y = y * jnp.minimum(2.0 / (1.0 + y + shifted), 1.0)
    col = jax.lax.broadcasted_iota(jnp.int32, y.shape, 1)
    return jnp.where(col == 0, 0.0, y)


def _body(scale_ref, x_hbm, o_hbm, xb, ob, in_sems, out_sems):
    inv_temp = scale_ref[0]

    def in_copy(i, slot):
        return pltpu.make_async_copy(
            x_hbm.at[pl.ds(i * _CH, _CH)], xb.at[slot], in_sems.at[slot])

    def out_copy(i, slot):
        return pltpu.make_async_copy(
            ob.at[slot], o_hbm.at[pl.ds(i * _CH, _CH)], out_sems.at[slot])

    for i in range(min(_IN_BUFS, _N)):
        in_copy(i, i).start()

    for i in range(_N):
        islot = i % _IN_BUFS
        oslot = i % _OUT_BUFS
        if i >= _OUT_BUFS:
            out_copy(i - _OUT_BUFS, oslot).wait()
        in_copy(i, islot).wait()
        ob[oslot] = _compute(xb[islot], inv_temp)
        out_copy(i, oslot).start()
        nxt = i + _IN_BUFS
        if nxt < _N:
            in_copy(nxt, islot).start()

    for i in range(_N - min(_OUT_BUFS, _N), _N):
        out_copy(i, i % _OUT_BUFS).wait()


@jax.jit
def kernel(scores, log_temperature):
    temp = jnp.clip(jnp.exp(log_temperature), 0.1, 10.0)
    inv_temp = (1.0 / temp).reshape(1).astype(jnp.float32)
    return pl.pallas_call(
        _body,
        in_specs=[
            pl.BlockSpec(memory_space=pltpu.SMEM),
            pl.BlockSpec(memory_space=pltpu.HBM),
        ],
        out_specs=pl.BlockSpec(memory_space=pltpu.HBM),
        out_shape=jax.ShapeDtypeStruct((_B, _T), jnp.float32),
        scratch_shapes=[
            pltpu.VMEM((_IN_BUFS, _CH, _T), jnp.float32),
            pltpu.VMEM((_OUT_BUFS, _CH, _T), jnp.float32),
            pltpu.SemaphoreType.DMA((_IN_BUFS,)),
            pltpu.SemaphoreType.DMA((_OUT_BUFS,)),
        ],
    )(inv_temp, scores)
